# baseline (speedup 1.0000x reference)
import functools

import jax
import jax.numpy as jnp
import numpy as np
from jax import lax
from jax.experimental import pallas as pl
from jax.experimental.pallas import tpu as pltpu
from jax.experimental.pallas import tpu_sc as plsc

RADIUS = 1
NNB = 27
FEAT = 128
GRID = 64
CENTER = 13

NC = 2
NS = 16
NW = NC * NS
CH = 96


def _shift_grid():
    r = np.arange(-RADIUS, RADIUS + 1)
    return np.stack(np.meshgrid(r, r, r, indexing="ij"), axis=-1).reshape(-1, 3)


POOL = 4


@functools.partial(jax.jit, static_argnums=(2,))
def _sc_gather(table, ids, n_ch):
    rows_per_w = n_ch * CH
    total = NW * rows_per_w
    n_groups = n_ch // POOL
    n_pairs = n_groups // 2

    def body(table_hbm, ids_hbm, out_hbm, idx_v, bufs, gsem, wsem):
        wid = lax.axis_index("s") * NC + lax.axis_index("c")
        base = wid * rows_per_w
        pltpu.sync_copy(ids_hbm.at[wid], idx_v)

        def start_g(j, b):
            pltpu.async_copy(table_hbm.at[idx_v.at[j]], bufs.at[b], gsem.at[b])

        def wait_g(b):
            pltpu.make_async_copy(
                table_hbm.at[idx_v.at[0]], bufs.at[b], gsem.at[b]).wait()

        def start_w(j, b):
            pltpu.async_copy(
                bufs.at[b], out_hbm.at[pl.ds(base + j * CH, CH)], wsem.at[b])

        def wait_w(b):
            pltpu.make_async_copy(
                bufs.at[b], out_hbm.at[pl.ds(base, CH)], wsem.at[b]).wait()

        A = list(range(POOL))
        B = list(range(POOL, 2 * POOL))

        for i, bb in enumerate(A):
            start_g(i, bb)
        for i, bb in enumerate(B):
            start_g(POOL + i, bb)

        def pair(t, carry):
            j0 = 2 * t * POOL
            for i, bb in enumerate(A):
                wait_g(bb)
                start_w(j0 + i, bb)
            for i, bb in enumerate(B):
                wait_g(bb)
                start_w(j0 + POOL + i, bb)
            for i, bb in enumerate(A):
                wait_w(bb)
                start_g(j0 + 2 * POOL + i, bb)
            for i, bb in enumerate(B):
                wait_w(bb)
                start_g(j0 + 3 * POOL + i, bb)
            return carry

        lax.fori_loop(0, n_pairs - 1, pair, 0, unroll=False)

        j0 = 2 * (n_pairs - 1) * POOL
        for i, bb in enumerate(A):
            wait_g(bb)
            start_w(j0 + i, bb)
        for i, bb in enumerate(B):
            wait_g(bb)
            start_w(j0 + POOL + i, bb)
        for bb in A + B:
            wait_w(bb)

    mesh = plsc.VectorSubcoreMesh(core_axis_name="c", subcore_axis_name="s")
    f = pl.kernel(
        body,
        out_type=jax.ShapeDtypeStruct((total, FEAT), jnp.float32),
        mesh=mesh,
        scratch_types=[
            pltpu.VMEM((n_ch, CH), jnp.int32),
            pltpu.VMEM((2 * POOL, CH, FEAT), jnp.float32),
            pltpu.SemaphoreType.DMA((2 * POOL,)),
            pltpu.SemaphoreType.DMA((2 * POOL,)),
        ],
    )
    return f(table, ids)


def _tc_matmul(gathered, mask, wt, b, block_n):
    n = gathered.shape[1]

    def body(g_ref, m_ref, wt_ref, b_ref, out_ref):
        m = m_ref[...]
        acc = jnp.broadcast_to(b_ref[...], (block_n, FEAT))
        for k in range(NNB):
            gk = (g_ref[k] * m[:, k:k + 1]).astype(jnp.bfloat16)
            acc = acc + jax.lax.dot_general(
                gk, wt_ref[k], (((1,), (0,)), ((), ())),
                preferred_element_type=jnp.float32)
        out_ref[...] = jnp.concatenate([g_ref[CENTER], acc], axis=1)

    return pl.pallas_call(
        body,
        grid=(n // block_n,),
        in_specs=[
            pl.BlockSpec((NNB, block_n, FEAT), lambda i: (0, i, 0)),
            pl.BlockSpec((block_n, NNB), lambda i: (i, 0)),
            pl.BlockSpec((NNB, FEAT, FEAT), lambda i: (0, 0, 0)),
            pl.BlockSpec((1, FEAT), lambda i: (0, 0)),
        ],
        out_specs=pl.BlockSpec((block_n, 2 * FEAT), lambda i: (i, 0)),
        out_shape=jax.ShapeDtypeStruct((n, 2 * FEAT), jnp.float32),
    )(gathered, mask, wt, b)


def kernel(query_indices, query_points, feature_volume, count_volume, W, b):
    del query_points, count_volume
    qi = query_indices.reshape(-1, 3)
    n = qi.shape[0]

    shift = jnp.asarray(_shift_grid(), dtype=jnp.int32)
    nb = qi[:, None, :] + shift[None, :, :]
    valid = jnp.all((nb >= 0) & (nb < GRID), axis=-1)
    nbc = jnp.clip(nb, 0, GRID - 1)
    ids = (nbc[..., 0] * GRID + nbc[..., 1]) * GRID + nbc[..., 2]
    mask = valid.astype(jnp.float32)

    table = feature_volume.reshape(GRID * GRID * GRID, FEAT)
    wt3 = W.T.reshape(NNB, FEAT, FEAT).astype(jnp.bfloat16)
    b2 = b.reshape(1, FEAT)
    ids_km = ids.T

    n_slices = 2
    ns = n // n_slices
    rows_per_w = NNB * ns // NW
    n_ch = rows_per_w // CH
    outs = []
    for s in range(n_slices):
        ids_s = ids_km[:, s * ns:(s + 1) * ns].reshape(NW, n_ch, CH)
        g_s = _sc_gather(table, ids_s, n_ch).reshape(NNB, ns, FEAT)
        m_s = mask[s * ns:(s + 1) * ns]
        outs.append(_tc_matmul(g_s, m_s, wt3, b2, block_n=512))
    out = jnp.concatenate(outs, axis=0)
    return (out, qi)


# speedup vs baseline: 1.0661x; 1.0661x over previous
import functools

import jax
import jax.numpy as jnp
import numpy as np
from jax import lax
from jax.experimental import pallas as pl
from jax.experimental.pallas import tpu as pltpu
from jax.experimental.pallas import tpu_sc as plsc

RADIUS = 1
NNB = 27
FEAT = 128
GRID = 64
CENTER = 13

NC = 2
NS = 16
NW = NC * NS
CH = 96


def _shift_grid():
    r = np.arange(-RADIUS, RADIUS + 1)
    return np.stack(np.meshgrid(r, r, r, indexing="ij"), axis=-1).reshape(-1, 3)


POOL = 4


@functools.partial(jax.jit, static_argnums=(2,))
def _sc_gather(table, ids, n_ch):
    rows_per_w = n_ch * CH
    total = NW * rows_per_w
    n_groups = n_ch // POOL
    n_pairs = n_groups // 2

    def body(table_hbm, ids_hbm, out_hbm, idx_v, bufs, gsem, wsem):
        wid = lax.axis_index("s") * NC + lax.axis_index("c")
        base = wid * rows_per_w
        pltpu.sync_copy(ids_hbm.at[wid], idx_v)

        def start_g(j, p, i):
            pltpu.async_copy(
                table_hbm.at[idx_v.at[j]],
                bufs.at[p].at[pl.ds(i * CH, CH)],
                gsem.at[p * POOL + i])

        def wait_g(p, i):
            pltpu.make_async_copy(
                table_hbm.at[idx_v.at[0]],
                bufs.at[p].at[pl.ds(i * CH, CH)],
                gsem.at[p * POOL + i]).wait()

        def start_w(j0, p):
            pltpu.async_copy(
                bufs.at[p],
                out_hbm.at[pl.ds(base + j0 * CH, POOL * CH)],
                wsem.at[p])

        def wait_w(p):
            pltpu.make_async_copy(
                bufs.at[p],
                out_hbm.at[pl.ds(base, POOL * CH)],
                wsem.at[p]).wait()

        for i in range(POOL):
            start_g(i, 0, i)
        for i in range(POOL):
            start_g(POOL + i, 1, i)

        def pair(t, carry):
            j0 = 2 * t * POOL
            for i in range(POOL):
                wait_g(0, i)
            start_w(j0, 0)
            for i in range(POOL):
                wait_g(1, i)
            start_w(j0 + POOL, 1)
            wait_w(0)
            for i in range(POOL):
                start_g(j0 + 2 * POOL + i, 0, i)
            wait_w(1)
            for i in range(POOL):
                start_g(j0 + 3 * POOL + i, 1, i)
            return carry

        lax.fori_loop(0, n_pairs - 1, pair, 0, unroll=False)

        j0 = 2 * (n_pairs - 1) * POOL
        for i in range(POOL):
            wait_g(0, i)
        start_w(j0, 0)
        for i in range(POOL):
            wait_g(1, i)
        start_w(j0 + POOL, 1)
        wait_w(0)
        wait_w(1)

    mesh = plsc.VectorSubcoreMesh(core_axis_name="c", subcore_axis_name="s")
    f = pl.kernel(
        body,
        out_type=jax.ShapeDtypeStruct((total, FEAT), jnp.float32),
        mesh=mesh,
        scratch_types=[
            pltpu.VMEM((n_ch, CH), jnp.int32),
            pltpu.VMEM((2, POOL * CH, FEAT), jnp.float32),
            pltpu.SemaphoreType.DMA((2 * POOL,)),
            pltpu.SemaphoreType.DMA((2,)),
        ],
    )
    return f(table, ids)


def _tc_matmul(gathered, mask, wt, b, block_n):
    n = gathered.shape[1]

    def body(g_ref, m_ref, wt_ref, b_ref, out_ref):
        m = m_ref[...]
        acc = jnp.broadcast_to(b_ref[...], (block_n, FEAT))
        for k in range(NNB):
            gk = (g_ref[k] * m[:, k:k + 1]).astype(jnp.bfloat16)
            acc = acc + jax.lax.dot_general(
                gk, wt_ref[k], (((1,), (0,)), ((), ())),
                preferred_element_type=jnp.float32)
        out_ref[...] = jnp.concatenate([g_ref[CENTER], acc], axis=1)

    return pl.pallas_call(
        body,
        grid=(n // block_n,),
        in_specs=[
            pl.BlockSpec((NNB, block_n, FEAT), lambda i: (0, i, 0)),
            pl.BlockSpec((block_n, NNB), lambda i: (i, 0)),
            pl.BlockSpec((NNB, FEAT, FEAT), lambda i: (0, 0, 0)),
            pl.BlockSpec((1, FEAT), lambda i: (0, 0)),
        ],
        out_specs=pl.BlockSpec((block_n, 2 * FEAT), lambda i: (i, 0)),
        out_shape=jax.ShapeDtypeStruct((n, 2 * FEAT), jnp.float32),
    )(gathered, mask, wt, b)


def kernel(query_indices, query_points, feature_volume, count_volume, W, b):
    del query_points, count_volume
    qi = query_indices.reshape(-1, 3)
    n = qi.shape[0]

    shift = jnp.asarray(_shift_grid(), dtype=jnp.int32)
    nb = qi[None, :, :] + shift[:, None, :]
    mask = jnp.all((nb >= 0) & (nb < GRID), axis=-1).T.astype(jnp.float32)
    nbc = jnp.clip(nb, 0, GRID - 1)
    ids = (nbc[..., 0] * GRID + nbc[..., 1]) * GRID + nbc[..., 2]

    table = feature_volume.reshape(GRID * GRID * GRID, FEAT)
    wt3 = W.T.reshape(NNB, FEAT, FEAT).astype(jnp.bfloat16)
    b2 = b.reshape(1, FEAT)
    ids_km = ids

    n_slices = 1
    ns = n // n_slices
    rows_per_w = NNB * ns // NW
    n_ch = rows_per_w // CH
    outs = []
    for s in range(n_slices):
        ids_s = ids_km[:, s * ns:(s + 1) * ns].reshape(NW, n_ch, CH)
        g_s = _sc_gather(table, ids_s, n_ch).reshape(NNB, ns, FEAT)
        m_s = mask[s * ns:(s + 1) * ns]
        outs.append(_tc_matmul(g_s, m_s, wt3, b2, block_n=1024))
    out = jnp.concatenate(outs, axis=0)
    return (out, qi)


# speedup vs baseline: 1.0814x; 1.0143x over previous
import functools

import jax
import jax.numpy as jnp
import numpy as np
from jax import lax
from jax.experimental import pallas as pl
from jax.experimental.pallas import tpu as pltpu
from jax.experimental.pallas import tpu_sc as plsc

RADIUS = 1
NNB = 27
FEAT = 128
GRID = 64
CENTER = 13

NC = 2
NS = 16
NW = NC * NS
CH = 128


def _shift_grid():
    r = np.arange(-RADIUS, RADIUS + 1)
    return np.stack(np.meshgrid(r, r, r, indexing="ij"), axis=-1).reshape(-1, 3)


POOL = 3


@functools.partial(jax.jit, static_argnums=(2,))
def _sc_gather(table, ids, n_ch):
    rows_per_w = n_ch * CH
    total = NW * rows_per_w
    n_groups = n_ch // POOL
    n_pairs = n_groups // 2

    def body(table_hbm, ids_hbm, out_hbm, idx_v, bufs, gsem, wsem):
        wid = lax.axis_index("s") * NC + lax.axis_index("c")
        base = wid * rows_per_w
        pltpu.sync_copy(ids_hbm.at[wid], idx_v)

        def start_g(j, p, i):
            pltpu.async_copy(
                table_hbm.at[idx_v.at[j]],
                bufs.at[p].at[pl.ds(i * CH, CH)],
                gsem.at[p * POOL + i])

        def wait_g(p, i):
            pltpu.make_async_copy(
                table_hbm.at[idx_v.at[0]],
                bufs.at[p].at[pl.ds(i * CH, CH)],
                gsem.at[p * POOL + i]).wait()

        def start_w(j0, p):
            pltpu.async_copy(
                bufs.at[p],
                out_hbm.at[pl.ds(base + j0 * CH, POOL * CH)],
                wsem.at[p])

        def wait_w(p):
            pltpu.make_async_copy(
                bufs.at[p],
                out_hbm.at[pl.ds(base, POOL * CH)],
                wsem.at[p]).wait()

        for i in range(POOL):
            start_g(i, 0, i)
        for i in range(POOL):
            start_g(POOL + i, 1, i)

        def pair(t, carry):
            j0 = 2 * t * POOL
            for i in range(POOL):
                wait_g(0, i)
            start_w(j0, 0)
            for i in range(POOL):
                wait_g(1, i)
            start_w(j0 + POOL, 1)
            wait_w(0)
            for i in range(POOL):
                start_g(j0 + 2 * POOL + i, 0, i)
            wait_w(1)
            for i in range(POOL):
                start_g(j0 + 3 * POOL + i, 1, i)
            return carry

        lax.fori_loop(0, n_pairs - 1, pair, 0, unroll=False)

        j0 = 2 * (n_pairs - 1) * POOL
        for i in range(POOL):
            wait_g(0, i)
        start_w(j0, 0)
        for i in range(POOL):
            wait_g(1, i)
        start_w(j0 + POOL, 1)
        wait_w(0)
        wait_w(1)

    mesh = plsc.VectorSubcoreMesh(core_axis_name="c", subcore_axis_name="s")
    f = pl.kernel(
        body,
        out_type=jax.ShapeDtypeStruct((total, FEAT), jnp.float32),
        mesh=mesh,
        scratch_types=[
            pltpu.VMEM((n_ch, CH), jnp.int32),
            pltpu.VMEM((2, POOL * CH, FEAT), jnp.float32),
            pltpu.SemaphoreType.DMA((2 * POOL,)),
            pltpu.SemaphoreType.DMA((2,)),
        ],
    )
    return f(table, ids)


def _tc_matmul(gathered, mask, wt, b, block_n):
    n = gathered.shape[1]

    def body(g_ref, m_ref, wt_ref, b_ref, out_ref):
        m = m_ref[...]
        acc = jnp.broadcast_to(b_ref[...], (block_n, FEAT))
        for k in range(NNB):
            gk = (g_ref[k] * m[:, k:k + 1]).astype(jnp.bfloat16)
            acc = acc + jax.lax.dot_general(
                gk, wt_ref[k], (((1,), (0,)), ((), ())),
                preferred_element_type=jnp.float32)
        out_ref[...] = jnp.concatenate([g_ref[CENTER], acc], axis=1)

    return pl.pallas_call(
        body,
        grid=(n // block_n,),
        in_specs=[
            pl.BlockSpec((NNB, block_n, FEAT), lambda i: (0, i, 0)),
            pl.BlockSpec((block_n, NNB), lambda i: (i, 0)),
            pl.BlockSpec((NNB, FEAT, FEAT), lambda i: (0, 0, 0)),
            pl.BlockSpec((1, FEAT), lambda i: (0, 0)),
        ],
        out_specs=pl.BlockSpec((block_n, 2 * FEAT), lambda i: (i, 0)),
        out_shape=jax.ShapeDtypeStruct((n, 2 * FEAT), jnp.float32),
    )(gathered, mask, wt, b)


def kernel(query_indices, query_points, feature_volume, count_volume, W, b):
    del query_points, count_volume
    qi = query_indices.reshape(-1, 3)
    n = qi.shape[0]

    shift = jnp.asarray(_shift_grid(), dtype=jnp.int32)
    nb = qi[None, :, :] + shift[:, None, :]
    mask = jnp.all((nb >= 0) & (nb < GRID), axis=-1).T.astype(jnp.float32)
    nbc = jnp.clip(nb, 0, GRID - 1)
    ids = (nbc[..., 0] * GRID + nbc[..., 1]) * GRID + nbc[..., 2]

    table = feature_volume.reshape(GRID * GRID * GRID, FEAT)
    wt3 = W.T.reshape(NNB, FEAT, FEAT).astype(jnp.bfloat16)
    b2 = b.reshape(1, FEAT)
    ids_km = ids

    n_slices = 1
    ns = n // n_slices
    rows_per_w = NNB * ns // NW
    n_ch = rows_per_w // CH
    outs = []
    for s in range(n_slices):
        ids_s = ids_km[:, s * ns:(s + 1) * ns].reshape(NW, n_ch, CH)
        g_s = _sc_gather(table, ids_s, n_ch).reshape(NNB, ns, FEAT)
        m_s = mask[s * ns:(s + 1) * ns]
        outs.append(_tc_matmul(g_s, m_s, wt3, b2, block_n=1024))
    out = jnp.concatenate(outs, axis=0)
    return (out, qi)
